# baseline (speedup 1.0000x reference)
import functools

import jax
import jax.numpy as jnp
import numpy as np
from jax import lax
from jax.experimental import pallas as pl
from jax.experimental.pallas import tpu as pltpu
from jax.experimental.pallas import tpu_sc as plsc

ANGLE = 30.0

_NC = 2
_NS = 16
_H = 512
_W = 512
_BC = 384
_BX = 128
_BY = 128
_RH = 192
_RW = 200
_GB = 4
_FPC = _BC // _NC


def _rect_origin(bx, by, np_mod):
    r0 = np_mod.minimum(np_mod.maximum(111 * bx + 64 * by - 100, 0), _H - _RH)
    c0 = np_mod.minimum(np_mod.maximum(-64 * bx + 111 * by + 93, 0), _W - _RW)
    return r0 & ~7, c0 & ~7


def _src_rows_cols(w, h):
    xx, yy = jnp.meshgrid(jnp.arange(w), jnp.arange(h), indexing="ij")
    xx = xx.astype(jnp.float32)
    yy = yy.astype(jnp.float32)
    xm, ym = (w + 1) / 2.0, (h + 1) / 2.0
    inds = jnp.concatenate(
        [(xx - xm).reshape(-1, 1), (yy - ym).reshape(-1, 1)], axis=1)
    a = jnp.array([ANGLE * np.pi / 180.0], dtype=jnp.float32)
    c = jnp.cos(a)[0]
    s = jnp.sin(a)[0]
    R = jnp.array([[c, s], [-s, c]], dtype=jnp.float32)
    inds = jnp.round(R @ inds.T) + jnp.array([[xm], [ym]], dtype=jnp.float32)
    inds = jnp.where(inds < 0, 0.0, inds)
    row0 = jnp.where(inds[0, :] >= w, w - 1.0, inds[0, :])
    row1 = jnp.where(inds[1, :] >= h, h - 1.0, inds[1, :])
    iinds = jnp.stack([row0, row1], axis=0).astype(jnp.int32)
    return iinds[0, :], iinds[1, :]


def _static_block_tables():
    k = np.arange(_H * _W, dtype=np.int64)
    x, y = k // _W, k % _W
    bx, by = x // _BX, y // _BY
    r0k, c0k = _rect_origin(bx, by, np)
    s = np.arange(_NS, dtype=np.int64)[:, None]
    j = np.arange(_BX * _BY, dtype=np.int64)[None, :]
    perm = ((s // _GB) * _BX + j // _BY) * _W + (s % _GB) * _BY + j % _BY
    return r0k.astype(np.int32), c0k.astype(np.int32), perm.reshape(-1)


_R0K, _C0K, _PERM = _static_block_tables()


@functools.partial(
    pl.kernel,
    out_type=jax.ShapeDtypeStruct((_BC, _H, _W), jnp.float32),
    mesh=plsc.VectorSubcoreMesh(
        core_axis_name="c", subcore_axis_name="s",
        num_cores=_NC, num_subcores=_NS),
    scratch_types=[
        pltpu.VMEM((_BX * _BY,), jnp.int32),
        pltpu.VMEM((_RH, _RW), jnp.float32),
        pltpu.VMEM((_RH, _RW), jnp.float32),
        pltpu.VMEM((_BX, _BY), jnp.float32),
        pltpu.VMEM((_BX, _BY), jnp.float32),
        pltpu.SemaphoreType.DMA,
        pltpu.SemaphoreType.DMA,
        pltpu.SemaphoreType.DMA,
        pltpu.SemaphoreType.DMA,
    ],
    compiler_params=pltpu.CompilerParams(
        use_tc_tiling_on_sc=False, needs_layout_passes=False),
)
def _sc_rotate(img_hbm, idx_hbm, out_hbm, idx_v, rect_a, rect_b,
               out_a, out_b, sem_ra, sem_rb, sem_oa, sem_ob):
    cid = lax.axis_index("c")
    sid = lax.axis_index("s")
    bx = sid // _GB
    by = sid % _GB
    r0, c0 = _rect_origin(bx, by, jnp)
    r0 = pl.multiple_of(r0, 8)
    c0 = pl.multiple_of(c0, 8)
    x0 = bx * _BX
    y0 = by * _BY
    fbase = cid * _FPC

    pltpu.sync_copy(idx_hbm.at[sid], idx_v)

    def rect_copy(f, buf, sem):
        return pltpu.make_async_copy(
            img_hbm.at[f, pl.ds(r0, _RH), pl.ds(c0, _RW)], buf, sem)

    def out_copy(f, buf, sem):
        return pltpu.make_async_copy(
            buf, out_hbm.at[f, pl.ds(x0, _BX), pl.ds(y0, _BY)], sem)

    def gather(rect, outbuf):
        def rowb(r, carry):
            for cb in range(_BY // 16):
                pk = idx_v[pl.ds(r * _BY + cb * 16, 16)]
                rl = pk >> 8
                cl = pk & 255
                outbuf[r, pl.ds(cb * 16, 16)] = plsc.load_gather(rect, [rl, cl])
            return carry
        lax.fori_loop(0, _BX, rowb, 0)

    rect_copy(fbase, rect_a, sem_ra).start()

    def body(g, carry):
        f = fbase + 2 * g
        rect_copy(f, rect_a, sem_ra).wait()
        rect_copy(f + 1, rect_b, sem_rb).start()

        @pl.when(g > 0)
        def _():
            out_copy(f - 2, out_a, sem_oa).wait()

        gather(rect_a, out_a)
        out_copy(f, out_a, sem_oa).start()

        rect_copy(f + 1, rect_b, sem_rb).wait()

        @pl.when(g < _FPC // 2 - 1)
        def _():
            rect_copy(f + 2, rect_a, sem_ra).start()

        @pl.when(g > 0)
        def _():
            out_copy(f - 1, out_b, sem_ob).wait()

        gather(rect_b, out_b)
        out_copy(f + 1, out_b, sem_ob).start()
        return carry

    lax.fori_loop(0, _FPC // 2, body, 0)
    out_copy(fbase + _FPC - 2, out_a, sem_oa).wait()
    out_copy(fbase + _FPC - 1, out_b, sem_ob).wait()


def kernel(img):
    w, h = img.shape[-2], img.shape[-1]
    src_r, src_c = _src_rows_cols(w, h)
    packed = (src_r - _R0K) * 256 + (src_c - _C0K)
    idx_blocks = packed[_PERM].reshape(_NS, _BX * _BY)
    out3 = _sc_rotate(img.reshape(_BC, _H, _W), idx_blocks)
    return out3.reshape(img.shape)


# speedup vs baseline: 1.0804x; 1.0804x over previous
import functools

import jax
import jax.numpy as jnp
import numpy as np
from jax import lax
from jax.experimental import pallas as pl
from jax.experimental.pallas import tpu as pltpu
from jax.experimental.pallas import tpu_sc as plsc

ANGLE = 30.0

_NC = 2
_NS = 16
_NW = _NC * _NS

_H = 512
_W = 512
_BC = 384
_B = _H * _W
_B_PER_W = _B // _NW
_CHUNK = 128
_NCHUNK = _B_PER_W // _CHUNK

_TBLK = 1024


def _flat_src_index(w, h):
    xx, yy = jnp.meshgrid(jnp.arange(w), jnp.arange(h), indexing="ij")
    xx = xx.astype(jnp.float32)
    yy = yy.astype(jnp.float32)
    xm, ym = (w + 1) / 2.0, (h + 1) / 2.0
    inds = jnp.concatenate(
        [(xx - xm).reshape(-1, 1), (yy - ym).reshape(-1, 1)], axis=1)
    a = jnp.array([ANGLE * np.pi / 180.0], dtype=jnp.float32)
    c = jnp.cos(a)[0]
    s = jnp.sin(a)[0]
    R = jnp.array([[c, s], [-s, c]], dtype=jnp.float32)
    inds = jnp.round(R @ inds.T) + jnp.array([[xm], [ym]], dtype=jnp.float32)
    inds = jnp.where(inds < 0, 0.0, inds)
    row0 = jnp.where(inds[0, :] >= w, w - 1.0, inds[0, :])
    row1 = jnp.where(inds[1, :] >= h, h - 1.0, inds[1, :])
    iinds = jnp.stack([row0, row1], axis=0).astype(jnp.int32)
    return iinds[0, :] * h + iinds[1, :]


def _transpose_kernel(x_ref, o_ref):
    o_ref[...] = x_ref[...].T


_fwd_transpose = pl.pallas_call(
    _transpose_kernel,
    grid=(_B // _TBLK,),
    in_specs=[pl.BlockSpec((_BC, _TBLK), lambda i: (0, i))],
    out_specs=pl.BlockSpec((_TBLK, _BC), lambda i: (i, 0)),
    out_shape=jax.ShapeDtypeStruct((_B, _BC), jnp.float32),
    compiler_params=pltpu.CompilerParams(
        dimension_semantics=("arbitrary",)),
)

_bwd_transpose = pl.pallas_call(
    _transpose_kernel,
    grid=(_B // _TBLK,),
    in_specs=[pl.BlockSpec((_TBLK, _BC), lambda i: (i, 0))],
    out_specs=pl.BlockSpec((_BC, _TBLK), lambda i: (0, i)),
    out_shape=jax.ShapeDtypeStruct((_BC, _B), jnp.float32),
    compiler_params=pltpu.CompilerParams(
        dimension_semantics=("arbitrary",)),
)


@functools.partial(
    pl.kernel,
    out_type=jax.ShapeDtypeStruct((_B, _BC), jnp.float32),
    mesh=plsc.VectorSubcoreMesh(
        core_axis_name="c", subcore_axis_name="s",
        num_cores=_NC, num_subcores=_NS),
    scratch_types=[
        pltpu.VMEM((_B_PER_W,), jnp.int32),
        pltpu.VMEM((_CHUNK, _BC), jnp.float32),
        pltpu.VMEM((_CHUNK, _BC), jnp.float32),
        pltpu.SemaphoreType.DMA,
        pltpu.SemaphoreType.DMA,
        pltpu.SemaphoreType.DMA,
        pltpu.SemaphoreType.DMA,
    ],
)
def _sc_gather(table_hbm, idx_hbm, out_hbm, idx_v, rows_a, rows_b,
               sem_ga, sem_gb, sem_oa, sem_ob):
    wid = lax.axis_index("s") * _NC + lax.axis_index("c")
    base = wid * _B_PER_W
    pltpu.sync_copy(idx_hbm.at[pl.ds(base, _B_PER_W)], idx_v)

    def gather_copy(i, buf, sem):
        return pltpu.make_async_copy(
            table_hbm.at[idx_v.at[pl.ds(i * _CHUNK, _CHUNK)]], buf, sem)

    def out_copy(i, buf, sem):
        return pltpu.make_async_copy(
            buf, out_hbm.at[pl.ds(base + i * _CHUNK, _CHUNK)], sem)

    gather_copy(0, rows_a, sem_ga).start()

    def body(g, carry):
        i = 2 * g
        gather_copy(i, rows_a, sem_ga).wait()
        gather_copy(i + 1, rows_b, sem_gb).start()

        @pl.when(g > 0)
        def _():
            out_copy(i - 2, rows_a, sem_oa).wait()

        out_copy(i, rows_a, sem_oa).start()
        gather_copy(i + 1, rows_b, sem_gb).wait()

        @pl.when(g < _NCHUNK // 2 - 1)
        def _():
            gather_copy(i + 2, rows_a, sem_ga).start()

        @pl.when(g > 0)
        def _():
            out_copy(i - 1, rows_b, sem_ob).wait()

        out_copy(i + 1, rows_b, sem_ob).start()
        return carry

    lax.fori_loop(0, _NCHUNK // 2, body, 0)
    out_copy(_NCHUNK - 2, rows_a, sem_oa).wait()
    out_copy(_NCHUNK - 1, rows_b, sem_ob).wait()


def kernel(img):
    w, h = img.shape[-2], img.shape[-1]
    src = _flat_src_index(w, h)
    table = _fwd_transpose(img.reshape(_BC, _B))
    out_t = _sc_gather(table, src)
    return _bwd_transpose(out_t).reshape(img.shape)


# speedup vs baseline: 1.9778x; 1.8306x over previous
import functools

import jax
import jax.numpy as jnp
import numpy as np
from jax import lax
from jax.experimental import pallas as pl
from jax.experimental.pallas import tpu as pltpu
from jax.experimental.pallas import tpu_sc as plsc

ANGLE = 30.0

_NC = 2
_NS = 16
_NW = _NC * _NS

_H = 512
_W = 512
_BC = 384
_B = _H * _W
_B_PER_W = _B // _NW
_CHUNK = 128
_NCHUNK = _B_PER_W // _CHUNK

_TBLK = 1024


def _flat_src_index(w, h):
    xx, yy = jnp.meshgrid(jnp.arange(w), jnp.arange(h), indexing="ij")
    xx = xx.astype(jnp.float32)
    yy = yy.astype(jnp.float32)
    xm, ym = (w + 1) / 2.0, (h + 1) / 2.0
    inds = jnp.concatenate(
        [(xx - xm).reshape(-1, 1), (yy - ym).reshape(-1, 1)], axis=1)
    a = jnp.array([ANGLE * np.pi / 180.0], dtype=jnp.float32)
    c = jnp.cos(a)[0]
    s = jnp.sin(a)[0]
    R = jnp.array([[c, s], [-s, c]], dtype=jnp.float32)
    inds = jnp.round(R @ inds.T) + jnp.array([[xm], [ym]], dtype=jnp.float32)
    inds = jnp.where(inds < 0, 0.0, inds)
    row0 = jnp.where(inds[0, :] >= w, w - 1.0, inds[0, :])
    row1 = jnp.where(inds[1, :] >= h, h - 1.0, inds[1, :])
    iinds = jnp.stack([row0, row1], axis=0).astype(jnp.int32)
    return iinds[0, :] * h + iinds[1, :]


_TROWS = 8


def _fwd_transpose_kernel(x_ref, o_ref):
    for r in range(_TROWS):
        o_ref[pl.ds(r * _W, _W), :] = x_ref[:, r, :].T


def _bwd_transpose_kernel(x_ref, o_ref):
    for r in range(_TROWS):
        o_ref[:, r, :] = x_ref[pl.ds(r * _W, _W), :].T


_fwd_transpose = pl.pallas_call(
    _fwd_transpose_kernel,
    grid=(_H // _TROWS,),
    in_specs=[pl.BlockSpec((_BC, _TROWS, _W), lambda i: (0, i, 0))],
    out_specs=pl.BlockSpec((_TROWS * _W, _BC), lambda i: (i, 0)),
    out_shape=jax.ShapeDtypeStruct((_B, _BC), jnp.float32),
    compiler_params=pltpu.CompilerParams(
        dimension_semantics=("arbitrary",)),
)

_bwd_transpose = pl.pallas_call(
    _bwd_transpose_kernel,
    grid=(_H // _TROWS,),
    in_specs=[pl.BlockSpec((_TROWS * _W, _BC), lambda i: (i, 0))],
    out_specs=pl.BlockSpec((_BC, _TROWS, _W), lambda i: (0, i, 0)),
    out_shape=jax.ShapeDtypeStruct((_BC, _H, _W), jnp.float32),
    compiler_params=pltpu.CompilerParams(
        dimension_semantics=("arbitrary",)),
)


@functools.partial(
    pl.kernel,
    out_type=jax.ShapeDtypeStruct((_B, _BC), jnp.float32),
    mesh=plsc.VectorSubcoreMesh(
        core_axis_name="c", subcore_axis_name="s",
        num_cores=_NC, num_subcores=_NS),
    scratch_types=[
        pltpu.VMEM((_B_PER_W,), jnp.int32),
        pltpu.VMEM((_CHUNK, _BC), jnp.float32),
        pltpu.VMEM((_CHUNK, _BC), jnp.float32),
        pltpu.SemaphoreType.DMA,
        pltpu.SemaphoreType.DMA,
        pltpu.SemaphoreType.DMA,
        pltpu.SemaphoreType.DMA,
    ],
)
def _sc_gather(table_hbm, idx_hbm, out_hbm, idx_v, rows_a, rows_b,
               sem_ga, sem_gb, sem_oa, sem_ob):
    wid = lax.axis_index("s") * _NC + lax.axis_index("c")
    base = wid * _B_PER_W
    pltpu.sync_copy(idx_hbm.at[pl.ds(base, _B_PER_W)], idx_v)

    def gather_copy(i, buf, sem):
        return pltpu.make_async_copy(
            table_hbm.at[idx_v.at[pl.ds(i * _CHUNK, _CHUNK)]], buf, sem)

    def out_copy(i, buf, sem):
        return pltpu.make_async_copy(
            buf, out_hbm.at[pl.ds(base + i * _CHUNK, _CHUNK)], sem)

    gather_copy(0, rows_a, sem_ga).start()

    def body(g, carry):
        i = 2 * g
        gather_copy(i, rows_a, sem_ga).wait()
        gather_copy(i + 1, rows_b, sem_gb).start()

        @pl.when(g > 0)
        def _():
            out_copy(i - 2, rows_a, sem_oa).wait()

        out_copy(i, rows_a, sem_oa).start()
        gather_copy(i + 1, rows_b, sem_gb).wait()

        @pl.when(g < _NCHUNK // 2 - 1)
        def _():
            gather_copy(i + 2, rows_a, sem_ga).start()

        @pl.when(g > 0)
        def _():
            out_copy(i - 1, rows_b, sem_ob).wait()

        out_copy(i + 1, rows_b, sem_ob).start()
        return carry

    lax.fori_loop(0, _NCHUNK // 2, body, 0)
    out_copy(_NCHUNK - 2, rows_a, sem_oa).wait()
    out_copy(_NCHUNK - 1, rows_b, sem_ob).wait()


def kernel(img):
    w, h = img.shape[-2], img.shape[-1]
    src = _flat_src_index(w, h)
    table = _fwd_transpose(img.reshape(_BC, _H, _W))
    out_t = _sc_gather(table, src)
    return _bwd_transpose(out_t).reshape(img.shape)


# speedup vs baseline: 2.0145x; 1.0186x over previous
import functools

import jax
import jax.numpy as jnp
import numpy as np
from jax import lax
from jax.experimental import pallas as pl
from jax.experimental.pallas import tpu as pltpu
from jax.experimental.pallas import tpu_sc as plsc

ANGLE = 30.0

_NC = 2
_NS = 16
_NW = _NC * _NS

_H = 512
_W = 512
_BC = 384
_B = _H * _W
_B_PER_W = _B // _NW
_CHUNK = 128
_NCHUNK = _B_PER_W // _CHUNK

_TBLK = 1024


def _flat_src_index(w, h):
    xx, yy = jnp.meshgrid(jnp.arange(w), jnp.arange(h), indexing="ij")
    xx = xx.astype(jnp.float32)
    yy = yy.astype(jnp.float32)
    xm, ym = (w + 1) / 2.0, (h + 1) / 2.0
    inds = jnp.concatenate(
        [(xx - xm).reshape(-1, 1), (yy - ym).reshape(-1, 1)], axis=1)
    a = jnp.array([ANGLE * np.pi / 180.0], dtype=jnp.float32)
    c = jnp.cos(a)[0]
    s = jnp.sin(a)[0]
    R = jnp.array([[c, s], [-s, c]], dtype=jnp.float32)
    inds = jnp.round(R @ inds.T) + jnp.array([[xm], [ym]], dtype=jnp.float32)
    inds = jnp.where(inds < 0, 0.0, inds)
    row0 = jnp.where(inds[0, :] >= w, w - 1.0, inds[0, :])
    row1 = jnp.where(inds[1, :] >= h, h - 1.0, inds[1, :])
    iinds = jnp.stack([row0, row1], axis=0).astype(jnp.int32)
    return iinds[0, :] * h + iinds[1, :]


_TROWS = 16


def _fwd_transpose_kernel(x_ref, o_ref):
    for r in range(_TROWS):
        o_ref[pl.ds(r * _W, _W), :] = x_ref[:, r, :].T


def _bwd_transpose_kernel(x_ref, o_ref):
    for r in range(_TROWS):
        o_ref[:, r, :] = x_ref[pl.ds(r * _W, _W), :].T


_fwd_transpose = pl.pallas_call(
    _fwd_transpose_kernel,
    grid=(_H // _TROWS,),
    in_specs=[pl.BlockSpec((_BC, _TROWS, _W), lambda i: (0, i, 0))],
    out_specs=pl.BlockSpec((_TROWS * _W, _BC), lambda i: (i, 0)),
    out_shape=jax.ShapeDtypeStruct((_B, _BC), jnp.float32),
    compiler_params=pltpu.CompilerParams(
        dimension_semantics=("arbitrary",)),
)

_bwd_transpose = pl.pallas_call(
    _bwd_transpose_kernel,
    grid=(_H // _TROWS,),
    in_specs=[pl.BlockSpec((_TROWS * _W, _BC), lambda i: (i, 0))],
    out_specs=pl.BlockSpec((_BC, _TROWS, _W), lambda i: (0, i, 0)),
    out_shape=jax.ShapeDtypeStruct((_BC, _H, _W), jnp.float32),
    compiler_params=pltpu.CompilerParams(
        dimension_semantics=("arbitrary",)),
)


@functools.partial(
    pl.kernel,
    out_type=jax.ShapeDtypeStruct((_B, _BC), jnp.float32),
    mesh=plsc.VectorSubcoreMesh(
        core_axis_name="c", subcore_axis_name="s",
        num_cores=_NC, num_subcores=_NS),
    scratch_types=[
        pltpu.VMEM((_B_PER_W,), jnp.int32),
        pltpu.VMEM((_CHUNK, _BC), jnp.float32),
        pltpu.VMEM((_CHUNK, _BC), jnp.float32),
        pltpu.SemaphoreType.DMA,
        pltpu.SemaphoreType.DMA,
        pltpu.SemaphoreType.DMA,
        pltpu.SemaphoreType.DMA,
    ],
)
def _sc_gather(table_hbm, idx_hbm, out_hbm, idx_v, rows_a, rows_b,
               sem_ga, sem_gb, sem_oa, sem_ob):
    wid = lax.axis_index("s") * _NC + lax.axis_index("c")
    base = wid * _B_PER_W
    pltpu.sync_copy(idx_hbm.at[pl.ds(base, _B_PER_W)], idx_v)

    def gather_copy(i, buf, sem):
        return pltpu.make_async_copy(
            table_hbm.at[idx_v.at[pl.ds(i * _CHUNK, _CHUNK)]], buf, sem)

    def out_copy(i, buf, sem):
        return pltpu.make_async_copy(
            buf, out_hbm.at[pl.ds(base + i * _CHUNK, _CHUNK)], sem)

    gather_copy(0, rows_a, sem_ga).start()

    def body(g, carry):
        i = 2 * g
        gather_copy(i, rows_a, sem_ga).wait()
        gather_copy(i + 1, rows_b, sem_gb).start()

        @pl.when(g > 0)
        def _():
            out_copy(i - 2, rows_a, sem_oa).wait()

        out_copy(i, rows_a, sem_oa).start()
        gather_copy(i + 1, rows_b, sem_gb).wait()

        @pl.when(g < _NCHUNK // 2 - 1)
        def _():
            gather_copy(i + 2, rows_a, sem_ga).start()

        @pl.when(g > 0)
        def _():
            out_copy(i - 1, rows_b, sem_ob).wait()

        out_copy(i + 1, rows_b, sem_ob).start()
        return carry

    lax.fori_loop(0, _NCHUNK // 2, body, 0)
    out_copy(_NCHUNK - 2, rows_a, sem_oa).wait()
    out_copy(_NCHUNK - 1, rows_b, sem_ob).wait()


def kernel(img):
    w, h = img.shape[-2], img.shape[-1]
    src = _flat_src_index(w, h)
    table = _fwd_transpose(img.reshape(_BC, _H, _W))
    out_t = _sc_gather(table, src)
    return _bwd_transpose(out_t).reshape(img.shape)
